# Initial kernel scaffold; baseline (speedup 1.0000x reference)
import functools
import math

import jax
import jax.numpy as jnp
from jax import lax
from jax.experimental import pallas as pl
from jax.experimental.pallas import tpu as pltpu
from jax.experimental.pallas import tpu_sc as plsc

N = 10000
E = 320000
NAB = 128
NF = 128
NI = 3
NRBF = 20
CUTOFF = 5.0
MAXZ = 100

_LN2 = math.log(2.0)


def _ssp(x):
    return jnp.maximum(x, 0.0) + jnp.log(1.0 + jnp.exp(-jnp.abs(x))) - _LN2


BE = 2000


def _filter_body(r_ref, w1_ref, b1_ref, w2_ref, b2_ref, out_ref):
    r = r_ref[...]
    d = jnp.sqrt(r[:, 0:1] ** 2 + r[:, 1:2] ** 2 + r[:, 2:3] ** 2)
    offs = jnp.linspace(0.0, CUTOFF, NRBF, dtype=jnp.float32).reshape(1, NRBF)
    width = CUTOFF / (NRBF - 1)
    coeff = -0.5 / (width * width)
    f = jnp.exp(coeff * (d - offs) ** 2)
    rcut = 0.5 * (jnp.cos(d * (math.pi / CUTOFF)) + 1.0)
    rcut = rcut * (d < CUTOFF).astype(jnp.float32)
    h = _ssp(jnp.dot(f, w1_ref[...], preferred_element_type=jnp.float32)
             + b1_ref[...])
    w = jnp.dot(h, w2_ref[...], preferred_element_type=jnp.float32) + b2_ref[...]
    out_ref[...] = w * rcut


def _filters(r_ij, fn_w1, fn_b1, fn_w2, fn_b2):
    grid = E // BE
    return pl.pallas_call(
        _filter_body,
        grid=(grid,),
        in_specs=[
            pl.BlockSpec((BE, 3), lambda i: (i, 0)),
            pl.BlockSpec((NRBF, NF), lambda i: (0, 0)),
            pl.BlockSpec((1, NF), lambda i: (0, 0)),
            pl.BlockSpec((NF, NF), lambda i: (0, 0)),
            pl.BlockSpec((1, NF), lambda i: (0, 0)),
        ],
        out_specs=pl.BlockSpec((BE, NF), lambda i: (i, 0)),
        out_shape=jax.ShapeDtypeStruct((E, NF), jnp.float32),
    )(r_ij, fn_w1, fn_b1.reshape(1, NF), fn_w2, fn_b2.reshape(1, NF))


BN = 2000


def _init_body(an_ref, emb_ref, w_ref, x_ref, xf_ref):
    ids = an_ref[0, 0, :].reshape(BN, 1)
    cols = lax.broadcasted_iota(jnp.int32, (BN, NAB), 1)
    onehot = (ids == cols).astype(jnp.float32)
    x = jnp.dot(onehot, emb_ref[...], preferred_element_type=jnp.float32)
    x_ref[...] = x
    xf_ref[...] = jnp.dot(x, w_ref[...], preferred_element_type=jnp.float32)


def _init(atomic_numbers, emb_pad, in2f_w0):
    an3 = atomic_numbers.astype(jnp.int32).reshape(N // BN, 1, BN)
    return pl.pallas_call(
        _init_body,
        grid=(N // BN,),
        in_specs=[
            pl.BlockSpec((1, 1, BN), lambda i: (i, 0, 0)),
            pl.BlockSpec((NAB, NAB), lambda i: (0, 0)),
            pl.BlockSpec((NAB, NF), lambda i: (0, 0)),
        ],
        out_specs=[
            pl.BlockSpec((BN, NAB), lambda i: (i, 0)),
            pl.BlockSpec((BN, NF), lambda i: (i, 0)),
        ],
        out_shape=[
            jax.ShapeDtypeStruct((N, NAB), jnp.float32),
            jax.ShapeDtypeStruct((N, NF), jnp.float32),
        ],
    )(an3, emb_pad, in2f_w0)


def _update_body(has_next, p_ref, x_ref, w1_ref, b1_ref, w2_ref, b2_ref,
                 wn_ref, x_out_ref, xf_out_ref=None):
    agg = p_ref[0] + p_ref[1]
    h = _ssp(jnp.dot(agg, w1_ref[...], preferred_element_type=jnp.float32)
             + b1_ref[...])
    v = jnp.dot(h, w2_ref[...], preferred_element_type=jnp.float32) + b2_ref[...]
    xn = x_ref[...] + v
    x_out_ref[...] = xn
    if has_next:
        xf_out_ref[...] = jnp.dot(xn, wn_ref[...],
                                  preferred_element_type=jnp.float32)


def _update(partials, x, w1, b1, w2, b2, in2f_next):
    has_next = in2f_next is not None
    wn = in2f_next if has_next else jnp.zeros((NAB, NF), jnp.float32)
    out_specs = [pl.BlockSpec((BN, NAB), lambda i: (i, 0))]
    out_shape = [jax.ShapeDtypeStruct((N, NAB), jnp.float32)]
    if has_next:
        out_specs.append(pl.BlockSpec((BN, NF), lambda i: (i, 0)))
        out_shape.append(jax.ShapeDtypeStruct((N, NF), jnp.float32))
    res = pl.pallas_call(
        functools.partial(_update_body, has_next),
        grid=(N // BN,),
        in_specs=[
            pl.BlockSpec((2, BN, NF), lambda i: (0, i, 0)),
            pl.BlockSpec((BN, NAB), lambda i: (i, 0)),
            pl.BlockSpec((NF, NAB), lambda i: (0, 0)),
            pl.BlockSpec((1, NAB), lambda i: (0, 0)),
            pl.BlockSpec((NAB, NAB), lambda i: (0, 0)),
            pl.BlockSpec((1, NAB), lambda i: (0, 0)),
            pl.BlockSpec((NAB, NF), lambda i: (0, 0)),
        ],
        out_specs=out_specs,
        out_shape=out_shape,
    )(partials, x, w1, b1.reshape(1, NAB), w2, b2.reshape(1, NAB), wn)
    if has_next:
        return res[0], res[1]
    return res[0], None


SC_CORES = 2
SC_SUBCORES = 16
CH = 128
EPC = E // SC_CORES
NCHC = EPC // CH
CH_BASE = NCHC // SC_SUBCORES
CH_REM = NCHC % SC_SUBCORES
RPT = N // SC_SUBCORES
ZR = 125


def _edge_body(xf_hbm, w_hbm, idxi_hbm, idxj_hbm, out_hbm,
               acc, idxi_v, idxj_v, w_v, rows_v, sem):
    cid = lax.axis_index("c")
    sid = lax.axis_index("s")

    def _zrow(i, carry):
        for j in range(NF // 16):
            w_v[i, pl.ds(j * 16, 16)] = jnp.zeros((16,), jnp.float32)
        return carry
    lax.fori_loop(0, ZR, _zrow, 0)
    for k in range(RPT // ZR):
        pltpu.sync_copy(w_v.at[pl.ds(0, ZR)],
                        acc.at[pl.ds(sid * RPT + k * ZR, ZR)])
    plsc.subcore_barrier()

    n_ch = jnp.where(sid < CH_REM, CH_BASE + 1, CH_BASE)
    ch0 = sid * CH_BASE + jnp.minimum(sid, CH_REM)
    ebase = cid * EPC + ch0 * CH

    def _chunk(k, carry):
        base = ebase + k * CH
        pltpu.sync_copy(idxj_hbm.at[pl.ds(base, CH)], idxj_v)
        pltpu.sync_copy(idxi_hbm.at[pl.ds(base, CH)], idxi_v)
        pltpu.sync_copy(w_hbm.at[pl.ds(base, CH)], w_v)
        pltpu.async_copy(xf_hbm.at[idxj_v], rows_v, sem).wait()

        def _mrow(i, c2):
            for j in range(NF // 16):
                s = pl.ds(j * 16, 16)
                rows_v[i, s] = rows_v[i, s] * w_v[i, s]
            return c2
        lax.fori_loop(0, CH, _mrow, 0)
        pltpu.sync_copy(rows_v, acc.at[idxi_v], add=True)
        return carry
    lax.fori_loop(0, n_ch, _chunk, 0)

    plsc.subcore_barrier()
    r0 = sid * RPT
    pltpu.sync_copy(acc.at[pl.ds(r0, RPT)],
                    out_hbm.at[pl.ds(cid * N + r0, RPT)])


def _edge_pass(xf, w_e, idx_i, idx_j):
    mesh = plsc.VectorSubcoreMesh(core_axis_name="c", subcore_axis_name="s")
    run = pl.kernel(
        _edge_body,
        out_type=jax.ShapeDtypeStruct((SC_CORES * N, NF), jnp.float32),
        mesh=mesh,
        scratch_types=[
            pltpu.VMEM_SHARED((N, NF), jnp.float32),
            pltpu.VMEM((CH,), jnp.int32),
            pltpu.VMEM((CH,), jnp.int32),
            pltpu.VMEM((CH, NF), jnp.float32),
            pltpu.VMEM((CH, NF), jnp.float32),
            pltpu.SemaphoreType.DMA,
        ],
    )
    out = run(xf, w_e, idx_i, idx_j)
    return out.reshape(SC_CORES, N, NF)


def kernel(atomic_numbers, r_ij, idx_i, idx_j, emb, in2f_W, fn_W1, fn_b1,
           fn_W2, fn_b2, f2out_W1, f2out_b1, f2out_W2, f2out_b2):
    emb_pad = jnp.pad(emb, ((0, NAB - MAXZ), (0, 0)))
    idx_i = idx_i.astype(jnp.int32)
    idx_j = idx_j.astype(jnp.int32)

    x, xf = _init(atomic_numbers, emb_pad, in2f_W[0])
    for t in range(NI):
        w_e = _filters(r_ij, fn_W1[t], fn_b1[t], fn_W2[t], fn_b2[t])
        partials = _edge_pass(xf, w_e, idx_i, idx_j)
        nxt = in2f_W[t + 1] if t + 1 < NI else None
        x, xf = _update(partials, x, f2out_W1[t], f2out_b1[t],
                        f2out_W2[t], f2out_b2[t], nxt)
    return x


# speedup vs baseline: 1.8650x; 1.8650x over previous
import functools
import math

import jax
import jax.numpy as jnp
from jax import lax
from jax.experimental import pallas as pl
from jax.experimental.pallas import tpu as pltpu
from jax.experimental.pallas import tpu_sc as plsc

N = 10000
E = 320000
NAB = 128
NF = 128
NI = 3
NRBF = 20
CUTOFF = 5.0
MAXZ = 100

_LN2 = math.log(2.0)


def _ssp(x):
    return jnp.maximum(x, 0.0) + jnp.log(1.0 + jnp.exp(-jnp.abs(x))) - _LN2


BE = 2000


def _filter_body(r_ref, w1_ref, b1_ref, w2_ref, b2_ref, out_ref):
    r = r_ref[...]
    d = jnp.sqrt(r[:, 0:1] ** 2 + r[:, 1:2] ** 2 + r[:, 2:3] ** 2)
    width = CUTOFF / (NRBF - 1)
    offs = lax.broadcasted_iota(jnp.int32, (1, NRBF), 1).astype(jnp.float32)
    offs = offs * width
    coeff = -0.5 / (width * width)
    f = jnp.exp(coeff * (d - offs) ** 2)
    rcut = 0.5 * (jnp.cos(d * (math.pi / CUTOFF)) + 1.0)
    rcut = rcut * (d < CUTOFF).astype(jnp.float32)
    h = _ssp(jnp.dot(f, w1_ref[...], preferred_element_type=jnp.float32)
             + b1_ref[...])
    w = jnp.dot(h, w2_ref[...], preferred_element_type=jnp.float32) + b2_ref[...]
    out_ref[...] = w * rcut


def _filters(r_ij, fn_w1, fn_b1, fn_w2, fn_b2):
    grid = E // BE
    return pl.pallas_call(
        _filter_body,
        grid=(grid,),
        in_specs=[
            pl.BlockSpec((BE, 3), lambda i: (i, 0)),
            pl.BlockSpec((NRBF, NF), lambda i: (0, 0)),
            pl.BlockSpec((1, NF), lambda i: (0, 0)),
            pl.BlockSpec((NF, NF), lambda i: (0, 0)),
            pl.BlockSpec((1, NF), lambda i: (0, 0)),
        ],
        out_specs=pl.BlockSpec((BE, NF), lambda i: (i, 0)),
        out_shape=jax.ShapeDtypeStruct((E, NF), jnp.float32),
    )(r_ij, fn_w1, fn_b1.reshape(1, NF), fn_w2, fn_b2.reshape(1, NF))


BN = 2000


def _init_body(an_ref, emb_ref, w_ref, x_ref, xf_ref):
    ids = an_ref[0, 0, :].reshape(BN, 1)
    cols = lax.broadcasted_iota(jnp.int32, (BN, NAB), 1)
    onehot = (ids == cols).astype(jnp.float32)
    x = jnp.dot(onehot, emb_ref[...], preferred_element_type=jnp.float32)
    x_ref[...] = x
    xf_ref[...] = jnp.dot(x, w_ref[...], preferred_element_type=jnp.float32)


def _init(atomic_numbers, emb_pad, in2f_w0):
    an3 = atomic_numbers.astype(jnp.int32).reshape(N // BN, 1, BN)
    return pl.pallas_call(
        _init_body,
        grid=(N // BN,),
        in_specs=[
            pl.BlockSpec((1, 1, BN), lambda i: (i, 0, 0)),
            pl.BlockSpec((NAB, NAB), lambda i: (0, 0)),
            pl.BlockSpec((NAB, NF), lambda i: (0, 0)),
        ],
        out_specs=[
            pl.BlockSpec((BN, NAB), lambda i: (i, 0)),
            pl.BlockSpec((BN, NF), lambda i: (i, 0)),
        ],
        out_shape=[
            jax.ShapeDtypeStruct((N, NAB), jnp.float32),
            jax.ShapeDtypeStruct((N, NF), jnp.float32),
        ],
    )(an3, emb_pad, in2f_w0)


def _update_body(has_next, p_ref, x_ref, w1_ref, b1_ref, w2_ref, b2_ref,
                 wn_ref, x_out_ref, xf_out_ref=None):
    agg = p_ref[0] + p_ref[1]
    h = _ssp(jnp.dot(agg, w1_ref[...], preferred_element_type=jnp.float32)
             + b1_ref[...])
    v = jnp.dot(h, w2_ref[...], preferred_element_type=jnp.float32) + b2_ref[...]
    xn = x_ref[...] + v
    x_out_ref[...] = xn
    if has_next:
        xf_out_ref[...] = jnp.dot(xn, wn_ref[...],
                                  preferred_element_type=jnp.float32)


def _update(partials, x, w1, b1, w2, b2, in2f_next):
    has_next = in2f_next is not None
    wn = in2f_next if has_next else jnp.zeros((NAB, NF), jnp.float32)
    out_specs = [pl.BlockSpec((BN, NAB), lambda i: (i, 0))]
    out_shape = [jax.ShapeDtypeStruct((N, NAB), jnp.float32)]
    if has_next:
        out_specs.append(pl.BlockSpec((BN, NF), lambda i: (i, 0)))
        out_shape.append(jax.ShapeDtypeStruct((N, NF), jnp.float32))
    res = pl.pallas_call(
        functools.partial(_update_body, has_next),
        grid=(N // BN,),
        in_specs=[
            pl.BlockSpec((2, BN, NF), lambda i: (0, i, 0)),
            pl.BlockSpec((BN, NAB), lambda i: (i, 0)),
            pl.BlockSpec((NF, NAB), lambda i: (0, 0)),
            pl.BlockSpec((1, NAB), lambda i: (0, 0)),
            pl.BlockSpec((NAB, NAB), lambda i: (0, 0)),
            pl.BlockSpec((1, NAB), lambda i: (0, 0)),
            pl.BlockSpec((NAB, NF), lambda i: (0, 0)),
        ],
        out_specs=out_specs,
        out_shape=out_shape,
    )(partials, x, w1, b1.reshape(1, NAB), w2, b2.reshape(1, NAB), wn)
    if has_next:
        return res[0], res[1]
    return res[0], None


SC_CORES = 2
SC_SUBCORES = 16
CH = 128
EPC = E // SC_CORES
NCHC = EPC // CH
CH_BASE = NCHC // SC_SUBCORES
CH_REM = NCHC % SC_SUBCORES
RPT = 624
RPT_LAST = N - 15 * RPT


def _edge_body(xf_hbm, w_hbm, idxi_hbm, idxj_hbm, out_hbm,
               acc, idxi_v, idxj_v, w_v, rows_v, sem):
    cid = lax.axis_index("c")
    sid = lax.axis_index("s")

    def _zrow(i, carry):
        for j in range(NF // 16):
            w_v[i, pl.ds(j * 16, 16)] = jnp.zeros((16,), jnp.float32)
        return carry
    lax.fori_loop(0, CH, _zrow, 0)
    r0 = sid * RPT
    for k in range(4):
        pltpu.sync_copy(w_v, acc.at[pl.ds(r0 + k * CH, CH)])

    @pl.when(sid == SC_SUBCORES - 1)
    def _zlast():
        pltpu.sync_copy(w_v, acc.at[pl.ds(r0 + 4 * CH, CH)])

    @pl.when(sid < SC_SUBCORES - 1)
    def _zrest():
        pltpu.sync_copy(w_v.at[pl.ds(0, RPT - 4 * CH)],
                        acc.at[pl.ds(r0 + 4 * CH, RPT - 4 * CH)])
    plsc.subcore_barrier()

    n_ch = jnp.where(sid < CH_REM, CH_BASE + 1, CH_BASE)
    ch0 = sid * CH_BASE + jnp.minimum(sid, CH_REM)
    ebase = cid * EPC + ch0 * CH

    def _chunk(k, carry):
        base = ebase + k * CH
        pltpu.sync_copy(idxj_hbm.at[pl.ds(base, CH)], idxj_v)
        pltpu.sync_copy(idxi_hbm.at[pl.ds(base, CH)], idxi_v)
        pltpu.sync_copy(w_hbm.at[pl.ds(base, CH)], w_v)
        pltpu.async_copy(xf_hbm.at[idxj_v], rows_v, sem).wait()

        def _mrow(i, c2):
            for j in range(NF // 16):
                s = pl.ds(j * 16, 16)
                rows_v[i, s] = rows_v[i, s] * w_v[i, s]
            return c2
        lax.fori_loop(0, CH, _mrow, 0)
        pltpu.sync_copy(rows_v, acc.at[idxi_v], add=True)
        return carry
    lax.fori_loop(0, n_ch, _chunk, 0)

    plsc.subcore_barrier()

    @pl.when(sid == SC_SUBCORES - 1)
    def _flast():
        pltpu.sync_copy(acc.at[pl.ds(r0, RPT_LAST)],
                        out_hbm.at[pl.ds(cid * N + r0, RPT_LAST)])

    @pl.when(sid < SC_SUBCORES - 1)
    def _frest():
        pltpu.sync_copy(acc.at[pl.ds(r0, RPT)],
                        out_hbm.at[pl.ds(cid * N + r0, RPT)])


def _edge_pass(xf, w_e, idx_i, idx_j):
    mesh = plsc.VectorSubcoreMesh(core_axis_name="c", subcore_axis_name="s")
    run = pl.kernel(
        _edge_body,
        out_type=jax.ShapeDtypeStruct((SC_CORES * N, NF), jnp.float32),
        mesh=mesh,
        scratch_types=[
            pltpu.VMEM_SHARED((N, NF), jnp.float32),
            pltpu.VMEM((CH,), jnp.int32),
            pltpu.VMEM((CH,), jnp.int32),
            pltpu.VMEM((CH, NF), jnp.float32),
            pltpu.VMEM((CH, NF), jnp.float32),
            pltpu.SemaphoreType.DMA,
        ],
    )
    out = run(xf, w_e, idx_i, idx_j)
    return out.reshape(SC_CORES, N, NF)


def kernel(atomic_numbers, r_ij, idx_i, idx_j, emb, in2f_W, fn_W1, fn_b1,
           fn_W2, fn_b2, f2out_W1, f2out_b1, f2out_W2, f2out_b2):
    emb_pad = jnp.pad(emb, ((0, NAB - MAXZ), (0, 0)))
    idx_i = idx_i.astype(jnp.int32)
    idx_j = idx_j.astype(jnp.int32)

    x, xf = _init(atomic_numbers, emb_pad, in2f_W[0])
    for t in range(NI):
        w_e = _filters(r_ij, fn_W1[t], fn_b1[t], fn_W2[t], fn_b2[t])
        partials = _edge_pass(xf, w_e, idx_i, idx_j)
        nxt = in2f_W[t + 1] if t + 1 < NI else None
        x, xf = _update(partials, x, f2out_W1[t], f2out_b1[t],
                        f2out_W2[t], f2out_b2[t], nxt)
    return x


# speedup vs baseline: 2.0216x; 1.0840x over previous
import functools
import math

import jax
import jax.numpy as jnp
from jax import lax
from jax.experimental import pallas as pl
from jax.experimental.pallas import tpu as pltpu
from jax.experimental.pallas import tpu_sc as plsc

N = 10000
E = 320000
NAB = 128
NF = 128
NI = 3
NRBF = 20
CUTOFF = 5.0
MAXZ = 100

_LN2 = math.log(2.0)


def _ssp(x):
    return jnp.maximum(x, 0.0) + jnp.log(1.0 + jnp.exp(-jnp.abs(x))) - _LN2


BE = 2000


def _filter_body(r_ref, w1_ref, b1_ref, w2_ref, b2_ref, out_ref):
    r = r_ref[...]
    d = jnp.sqrt(r[:, 0:1] ** 2 + r[:, 1:2] ** 2 + r[:, 2:3] ** 2)
    width = CUTOFF / (NRBF - 1)
    offs = lax.broadcasted_iota(jnp.int32, (1, NRBF), 1).astype(jnp.float32)
    offs = offs * width
    coeff = -0.5 / (width * width)
    f = jnp.exp(coeff * (d - offs) ** 2)
    rcut = 0.5 * (jnp.cos(d * (math.pi / CUTOFF)) + 1.0)
    rcut = rcut * (d < CUTOFF).astype(jnp.float32)
    h = _ssp(jnp.dot(f, w1_ref[...], preferred_element_type=jnp.float32)
             + b1_ref[...])
    w = jnp.dot(h, w2_ref[...], preferred_element_type=jnp.float32) + b2_ref[...]
    out_ref[...] = w * rcut


def _filters(r_ij, fn_w1, fn_b1, fn_w2, fn_b2):
    grid = E // BE
    return pl.pallas_call(
        _filter_body,
        grid=(grid,),
        in_specs=[
            pl.BlockSpec((BE, 3), lambda i: (i, 0)),
            pl.BlockSpec((NRBF, NF), lambda i: (0, 0)),
            pl.BlockSpec((1, NF), lambda i: (0, 0)),
            pl.BlockSpec((NF, NF), lambda i: (0, 0)),
            pl.BlockSpec((1, NF), lambda i: (0, 0)),
        ],
        out_specs=pl.BlockSpec((BE, NF), lambda i: (i, 0)),
        out_shape=jax.ShapeDtypeStruct((E, NF), jnp.float32),
    )(r_ij, fn_w1, fn_b1.reshape(1, NF), fn_w2, fn_b2.reshape(1, NF))


BN = 2000


def _init_body(an_ref, emb_ref, w_ref, x_ref, xf_ref):
    ids = an_ref[0, 0, :].reshape(BN, 1)
    cols = lax.broadcasted_iota(jnp.int32, (BN, NAB), 1)
    onehot = (ids == cols).astype(jnp.float32)
    x = jnp.dot(onehot, emb_ref[...], preferred_element_type=jnp.float32)
    x_ref[...] = x
    xf_ref[...] = jnp.dot(x, w_ref[...], preferred_element_type=jnp.float32)


def _init(atomic_numbers, emb_pad, in2f_w0):
    an3 = atomic_numbers.astype(jnp.int32).reshape(N // BN, 1, BN)
    return pl.pallas_call(
        _init_body,
        grid=(N // BN,),
        in_specs=[
            pl.BlockSpec((1, 1, BN), lambda i: (i, 0, 0)),
            pl.BlockSpec((NAB, NAB), lambda i: (0, 0)),
            pl.BlockSpec((NAB, NF), lambda i: (0, 0)),
        ],
        out_specs=[
            pl.BlockSpec((BN, NAB), lambda i: (i, 0)),
            pl.BlockSpec((BN, NF), lambda i: (i, 0)),
        ],
        out_shape=[
            jax.ShapeDtypeStruct((N, NAB), jnp.float32),
            jax.ShapeDtypeStruct((N, NF), jnp.float32),
        ],
    )(an3, emb_pad, in2f_w0)


def _update_body(has_next, p_ref, x_ref, w1_ref, b1_ref, w2_ref, b2_ref,
                 wn_ref, x_out_ref, xf_out_ref=None):
    agg = p_ref[0] + p_ref[1]
    h = _ssp(jnp.dot(agg, w1_ref[...], preferred_element_type=jnp.float32)
             + b1_ref[...])
    v = jnp.dot(h, w2_ref[...], preferred_element_type=jnp.float32) + b2_ref[...]
    xn = x_ref[...] + v
    x_out_ref[...] = xn
    if has_next:
        xf_out_ref[...] = jnp.dot(xn, wn_ref[...],
                                  preferred_element_type=jnp.float32)


def _update(partials, x, w1, b1, w2, b2, in2f_next):
    has_next = in2f_next is not None
    wn = in2f_next if has_next else jnp.zeros((NAB, NF), jnp.float32)
    out_specs = [pl.BlockSpec((BN, NAB), lambda i: (i, 0))]
    out_shape = [jax.ShapeDtypeStruct((N, NAB), jnp.float32)]
    if has_next:
        out_specs.append(pl.BlockSpec((BN, NF), lambda i: (i, 0)))
        out_shape.append(jax.ShapeDtypeStruct((N, NF), jnp.float32))
    res = pl.pallas_call(
        functools.partial(_update_body, has_next),
        grid=(N // BN,),
        in_specs=[
            pl.BlockSpec((2, BN, NF), lambda i: (0, i, 0)),
            pl.BlockSpec((BN, NAB), lambda i: (i, 0)),
            pl.BlockSpec((NF, NAB), lambda i: (0, 0)),
            pl.BlockSpec((1, NAB), lambda i: (0, 0)),
            pl.BlockSpec((NAB, NAB), lambda i: (0, 0)),
            pl.BlockSpec((1, NAB), lambda i: (0, 0)),
            pl.BlockSpec((NAB, NF), lambda i: (0, 0)),
        ],
        out_specs=out_specs,
        out_shape=out_shape,
    )(partials, x, w1, b1.reshape(1, NAB), w2, b2.reshape(1, NAB), wn)
    if has_next:
        return res[0], res[1]
    return res[0], None


SC_CORES = 2
SC_SUBCORES = 16
CH = 64
EPC = E // SC_CORES
NCHC = EPC // CH
CH_BASE = NCHC // SC_SUBCORES
CH_REM = NCHC % SC_SUBCORES
RPT = 624
RPT_LAST = N - 15 * RPT
NUNROLL = 6
NGROUPS = CH_BASE // NUNROLL


def _edge_body(xf_hbm, w_hbm, idxi_hbm, idxj_hbm, out_hbm,
               acc, idxi_v, idxj_v, w_v, rows_v,
               semi0, semi1, semi2, semw0, semw1, semg0, semg1):
    semi = (semi0, semi1, semi2)
    semw = (semw0, semw1)
    semg = (semg0, semg1)
    cid = lax.axis_index("c")
    sid = lax.axis_index("s")

    zbuf = w_v.at[0]

    def _zrow(i, carry):
        for j in range(NF // 16):
            zbuf[i, pl.ds(j * 16, 16)] = jnp.zeros((16,), jnp.float32)
        return carry
    lax.fori_loop(0, CH, _zrow, 0)
    r0 = sid * RPT
    for k in range(RPT // CH):
        pltpu.sync_copy(zbuf, acc.at[pl.ds(r0 + k * CH, CH)])

    @pl.when(sid == SC_SUBCORES - 1)
    def _zlast():
        pltpu.sync_copy(zbuf, acc.at[pl.ds(r0 + (RPT // CH) * CH, CH)])

    @pl.when(sid < SC_SUBCORES - 1)
    def _zrest():
        zr = RPT - (RPT // CH) * CH
        pltpu.sync_copy(zbuf.at[pl.ds(0, zr)],
                        acc.at[pl.ds(r0 + (RPT // CH) * CH, zr)])
    plsc.subcore_barrier()

    ebase = cid * EPC + sid * (CH_BASE * CH)

    def _issue_idx(base, i3):
        pltpu.async_copy(idxj_hbm.at[pl.ds(base, CH)], idxj_v.at[i3],
                         semi[i3])
        pltpu.async_copy(idxi_hbm.at[pl.ds(base, CH)], idxi_v.at[i3],
                         semi[i3])

    def _wait_idx(i3):
        pltpu.make_async_copy(idxj_hbm.at[pl.ds(0, CH)], idxj_v.at[i3],
                              semi[i3]).wait()
        pltpu.make_async_copy(idxi_hbm.at[pl.ds(0, CH)], idxi_v.at[i3],
                              semi[i3]).wait()

    def _issue_w(base, i2):
        pltpu.async_copy(w_hbm.at[pl.ds(base, CH)], w_v.at[i2], semw[i2])

    def _wait_w(i2):
        pltpu.make_async_copy(w_hbm.at[pl.ds(0, CH)], w_v.at[i2],
                              semw[i2]).wait()

    def _issue_g(i3, i2):
        pltpu.async_copy(xf_hbm.at[idxj_v.at[i3]], rows_v.at[i2], semg[i2])

    def _wait_g(i3, i2):
        pltpu.make_async_copy(xf_hbm.at[idxj_v.at[i3]], rows_v.at[i2],
                              semg[i2]).wait()

    def _mul_scatter(i3, i2):
        def _mrow(i, cc):
            for j in range(NF // 16):
                s = pl.ds(j * 16, 16)
                rows_v[i2, i, s] = rows_v[i2, i, s] * w_v[i2, i, s]
            return cc
        lax.fori_loop(0, CH, _mrow, 0)
        pltpu.sync_copy(rows_v.at[i2], acc.at[idxi_v.at[i3]], add=True)

    _issue_idx(ebase, 0)
    _issue_idx(ebase + CH, 1)
    _wait_idx(0)
    _issue_g(0, 0)
    _issue_w(ebase, 0)

    def _group(g, carry):
        gbase = ebase + g * (NUNROLL * CH)
        for u in range(NUNROLL):
            i3, i3n = u % 3, (u + 1) % 3
            i2, i2n = u % 2, (u + 1) % 2
            kbase = gbase + u * CH
            if u < NUNROLL - 2:
                _issue_idx(kbase + 2 * CH, (u + 2) % 3)
            else:
                @pl.when(g < NGROUPS - 1)
                def _a():
                    _issue_idx(kbase + 2 * CH, (u + 2) % 3)
            if u < NUNROLL - 1:
                _wait_idx(i3n)
                _issue_g(i3n, i2n)
                _issue_w(kbase + CH, i2n)
            else:
                @pl.when(g < NGROUPS - 1)
                def _b():
                    _wait_idx(i3n)
                    _issue_g(i3n, i2n)
                    _issue_w(kbase + CH, i2n)
            _wait_g(i3, i2)
            _wait_w(i2)
            _mul_scatter(i3, i2)
        return carry
    lax.fori_loop(0, NGROUPS, _group, 0)

    @pl.when(sid < CH_REM)
    def _extra():
        base = cid * EPC + (SC_SUBCORES * CH_BASE + sid) * CH
        pltpu.sync_copy(idxj_hbm.at[pl.ds(base, CH)], idxj_v.at[0])
        pltpu.sync_copy(idxi_hbm.at[pl.ds(base, CH)], idxi_v.at[0])
        pltpu.sync_copy(w_hbm.at[pl.ds(base, CH)], w_v.at[0])
        pltpu.async_copy(xf_hbm.at[idxj_v.at[0]], rows_v.at[0],
                         semg[0]).wait()
        _mul_scatter(0, 0)

    plsc.subcore_barrier()

    @pl.when(sid == SC_SUBCORES - 1)
    def _flast():
        pltpu.sync_copy(acc.at[pl.ds(r0, RPT_LAST)],
                        out_hbm.at[pl.ds(cid * N + r0, RPT_LAST)])

    @pl.when(sid < SC_SUBCORES - 1)
    def _frest():
        pltpu.sync_copy(acc.at[pl.ds(r0, RPT)],
                        out_hbm.at[pl.ds(cid * N + r0, RPT)])


def _edge_pass(xf, w_e, idx_i, idx_j):
    mesh = plsc.VectorSubcoreMesh(core_axis_name="c", subcore_axis_name="s")
    run = pl.kernel(
        _edge_body,
        out_type=jax.ShapeDtypeStruct((SC_CORES * N, NF), jnp.float32),
        mesh=mesh,
        scratch_types=[
            pltpu.VMEM_SHARED((N, NF), jnp.float32),
            pltpu.VMEM((3, CH), jnp.int32),
            pltpu.VMEM((3, CH), jnp.int32),
            pltpu.VMEM((2, CH, NF), jnp.float32),
            pltpu.VMEM((2, CH, NF), jnp.float32),
        ] + [pltpu.SemaphoreType.DMA] * 7,
    )
    out = run(xf, w_e, idx_i, idx_j)
    return out.reshape(SC_CORES, N, NF)


def kernel(atomic_numbers, r_ij, idx_i, idx_j, emb, in2f_W, fn_W1, fn_b1,
           fn_W2, fn_b2, f2out_W1, f2out_b1, f2out_W2, f2out_b2):
    emb_pad = jnp.pad(emb, ((0, NAB - MAXZ), (0, 0)))
    idx_i = idx_i.astype(jnp.int32)
    idx_j = idx_j.astype(jnp.int32)

    x, xf = _init(atomic_numbers, emb_pad, in2f_W[0])
    for t in range(NI):
        w_e = _filters(r_ij, fn_W1[t], fn_b1[t], fn_W2[t], fn_b2[t])
        partials = _edge_pass(xf, w_e, idx_i, idx_j)
        nxt = in2f_W[t + 1] if t + 1 < NI else None
        x, xf = _update(partials, x, f2out_W1[t], f2out_b1[t],
                        f2out_W2[t], f2out_b2[t], nxt)
    return x


# speedup vs baseline: 5.0771x; 2.5114x over previous
import functools
import math

import jax
import jax.numpy as jnp
from jax import lax
from jax.experimental import pallas as pl
from jax.experimental.pallas import tpu as pltpu
from jax.experimental.pallas import tpu_sc as plsc

N = 10000
E = 320000
NAB = 128
NF = 128
NI = 3
NRBF = 20
CUTOFF = 5.0
MAXZ = 100

_LN2 = math.log(2.0)


def _ssp(x):
    return jnp.maximum(x, 0.0) + jnp.log(1.0 + jnp.exp(-jnp.abs(x))) - _LN2


TKNOT = 16384
TPAD = 2048
TROWS = TKNOT + TPAD
TB = 2048
DH = CUTOFF / TKNOT


def _table_body(w1_ref, b1_ref, w2_ref, b2_ref, out_ref):
    i = pl.program_id(0)
    row = lax.broadcasted_iota(jnp.int32, (TB, 1), 0) + i * TB
    d = row.astype(jnp.float32) * DH
    width = CUTOFF / (NRBF - 1)
    offs = lax.broadcasted_iota(jnp.int32, (1, NRBF), 1).astype(jnp.float32)
    offs = offs * width
    coeff = -0.5 / (width * width)
    f = jnp.exp(coeff * (d - offs) ** 2)
    rcut = 0.5 * (jnp.cos(d * (math.pi / CUTOFF)) + 1.0)
    rcut = rcut * (d < CUTOFF).astype(jnp.float32)
    hid = _ssp(jnp.dot(f, w1_ref[...], preferred_element_type=jnp.float32)
               + b1_ref[...])
    w = (jnp.dot(hid, w2_ref[...], preferred_element_type=jnp.float32)
         + b2_ref[...])
    out_ref[...] = w * rcut


def _filter_table(fn_w1, fn_b1, fn_w2, fn_b2):
    return pl.pallas_call(
        _table_body,
        grid=(TROWS // TB,),
        in_specs=[
            pl.BlockSpec((NRBF, NF), lambda i: (0, 0)),
            pl.BlockSpec((1, NF), lambda i: (0, 0)),
            pl.BlockSpec((NF, NF), lambda i: (0, 0)),
            pl.BlockSpec((1, NF), lambda i: (0, 0)),
        ],
        out_specs=pl.BlockSpec((TB, NF), lambda i: (i, 0)),
        out_shape=jax.ShapeDtypeStruct((TROWS, NF), jnp.float32),
    )(fn_w1, fn_b1.reshape(1, NF), fn_w2, fn_b2.reshape(1, NF))


BE = 2000


def _prep_body(r_ref, out_ref):
    r = r_ref[...]
    d = jnp.sqrt(r[:, 0:1] ** 2 + r[:, 1:2] ** 2 + r[:, 2:3] ** 2)
    k = (d * (1.0 / DH) + 0.5).astype(jnp.int32)
    k = jnp.minimum(k, TROWS - 1)
    out_ref[...] = k.reshape(1, 1, BE)


def _prep(r_ij):
    out = pl.pallas_call(
        _prep_body,
        grid=(E // BE,),
        in_specs=[pl.BlockSpec((BE, 3), lambda i: (i, 0))],
        out_specs=pl.BlockSpec((1, 1, BE), lambda i: (i, 0, 0)),
        out_shape=jax.ShapeDtypeStruct((E // BE, 1, BE), jnp.int32),
    )(r_ij)
    return out.reshape(E)


BN = 2000


def _init_body(an_ref, emb_ref, w_ref, x_ref, xf_ref):
    ids = an_ref[0, 0, :].reshape(BN, 1)
    cols = lax.broadcasted_iota(jnp.int32, (BN, NAB), 1)
    onehot = (ids == cols).astype(jnp.float32)
    x = jnp.dot(onehot, emb_ref[...], preferred_element_type=jnp.float32)
    x_ref[...] = x
    xf_ref[...] = jnp.dot(x, w_ref[...], preferred_element_type=jnp.float32)


def _init(atomic_numbers, emb_pad, in2f_w0):
    an3 = atomic_numbers.astype(jnp.int32).reshape(N // BN, 1, BN)
    return pl.pallas_call(
        _init_body,
        grid=(N // BN,),
        in_specs=[
            pl.BlockSpec((1, 1, BN), lambda i: (i, 0, 0)),
            pl.BlockSpec((NAB, NAB), lambda i: (0, 0)),
            pl.BlockSpec((NAB, NF), lambda i: (0, 0)),
        ],
        out_specs=[
            pl.BlockSpec((BN, NAB), lambda i: (i, 0)),
            pl.BlockSpec((BN, NF), lambda i: (i, 0)),
        ],
        out_shape=[
            jax.ShapeDtypeStruct((N, NAB), jnp.float32),
            jax.ShapeDtypeStruct((N, NF), jnp.float32),
        ],
    )(an3, emb_pad, in2f_w0)


def _update_body(has_next, p_ref, x_ref, w1_ref, b1_ref, w2_ref, b2_ref,
                 wn_ref, x_out_ref, xf_out_ref=None):
    agg = p_ref[0] + p_ref[1]
    h = _ssp(jnp.dot(agg, w1_ref[...], preferred_element_type=jnp.float32)
             + b1_ref[...])
    v = jnp.dot(h, w2_ref[...], preferred_element_type=jnp.float32) + b2_ref[...]
    xn = x_ref[...] + v
    x_out_ref[...] = xn
    if has_next:
        xf_out_ref[...] = jnp.dot(xn, wn_ref[...],
                                  preferred_element_type=jnp.float32)


def _update(partials, x, w1, b1, w2, b2, in2f_next):
    has_next = in2f_next is not None
    wn = in2f_next if has_next else jnp.zeros((NAB, NF), jnp.float32)
    out_specs = [pl.BlockSpec((BN, NAB), lambda i: (i, 0))]
    out_shape = [jax.ShapeDtypeStruct((N, NAB), jnp.float32)]
    if has_next:
        out_specs.append(pl.BlockSpec((BN, NF), lambda i: (i, 0)))
        out_shape.append(jax.ShapeDtypeStruct((N, NF), jnp.float32))
    res = pl.pallas_call(
        functools.partial(_update_body, has_next),
        grid=(N // BN,),
        in_specs=[
            pl.BlockSpec((2, BN, NF), lambda i: (0, i, 0)),
            pl.BlockSpec((BN, NAB), lambda i: (i, 0)),
            pl.BlockSpec((NF, NAB), lambda i: (0, 0)),
            pl.BlockSpec((1, NAB), lambda i: (0, 0)),
            pl.BlockSpec((NAB, NAB), lambda i: (0, 0)),
            pl.BlockSpec((1, NAB), lambda i: (0, 0)),
            pl.BlockSpec((NAB, NF), lambda i: (0, 0)),
        ],
        out_specs=out_specs,
        out_shape=out_shape,
    )(partials, x, w1, b1.reshape(1, NAB), w2, b2.reshape(1, NAB), wn)
    if has_next:
        return res[0], res[1]
    return res[0], None


SC_CORES = 2
SC_SUBCORES = 16
CH = 64
EPC = E // SC_CORES
NCHC = EPC // CH
CH_BASE = NCHC // SC_SUBCORES
CH_REM = NCHC % SC_SUBCORES
RPT = 624
RPT_LAST = N - 15 * RPT
NUNROLL = 6
NGROUPS = CH_BASE // NUNROLL


def _edge_body(xf_hbm, gt_hbm, idxi_hbm, idxj_hbm, idxk_hbm, out_hbm,
               acc, idxi_v, idxj_v, idxk_v, g_v, rows_v,
               semi0, semi1, semi2, semw0, semw1, semg0, semg1):
    semi = (semi0, semi1, semi2)
    semw = (semw0, semw1)
    semg = (semg0, semg1)
    cid = lax.axis_index("c")
    sid = lax.axis_index("s")

    zbuf = g_v.at[0]

    def _zrow(i, carry):
        for j in range(NF // 16):
            zbuf[i, pl.ds(j * 16, 16)] = jnp.zeros((16,), jnp.float32)
        return carry
    lax.fori_loop(0, CH, _zrow, 0)
    r0 = sid * RPT
    for k in range(RPT // CH):
        pltpu.sync_copy(zbuf, acc.at[pl.ds(r0 + k * CH, CH)])

    @pl.when(sid == SC_SUBCORES - 1)
    def _zlast():
        pltpu.sync_copy(zbuf, acc.at[pl.ds(r0 + (RPT // CH) * CH, CH)])

    @pl.when(sid < SC_SUBCORES - 1)
    def _zrest():
        zr = RPT - (RPT // CH) * CH
        pltpu.sync_copy(zbuf.at[pl.ds(0, zr)],
                        acc.at[pl.ds(r0 + (RPT // CH) * CH, zr)])
    plsc.subcore_barrier()

    ebase = cid * EPC + sid * (CH_BASE * CH)

    def _issue_idx(base, i3):
        pltpu.async_copy(idxj_hbm.at[pl.ds(base, CH)], idxj_v.at[i3],
                         semi[i3])
        pltpu.async_copy(idxi_hbm.at[pl.ds(base, CH)], idxi_v.at[i3],
                         semi[i3])
        pltpu.async_copy(idxk_hbm.at[pl.ds(base, CH)], idxk_v.at[i3],
                         semi[i3])

    def _wait_idx(i3):
        pltpu.make_async_copy(idxj_hbm.at[pl.ds(0, CH)], idxj_v.at[i3],
                              semi[i3]).wait()
        pltpu.make_async_copy(idxi_hbm.at[pl.ds(0, CH)], idxi_v.at[i3],
                              semi[i3]).wait()
        pltpu.make_async_copy(idxk_hbm.at[pl.ds(0, CH)], idxk_v.at[i3],
                              semi[i3]).wait()

    def _issue_gt(i3, i2):
        pltpu.async_copy(gt_hbm.at[idxk_v.at[i3]], g_v.at[i2], semw[i2])

    def _wait_gt(i3, i2):
        pltpu.make_async_copy(gt_hbm.at[idxk_v.at[i3]], g_v.at[i2],
                              semw[i2]).wait()

    def _issue_g(i3, i2):
        pltpu.async_copy(xf_hbm.at[idxj_v.at[i3]], rows_v.at[i2], semg[i2])

    def _wait_g(i3, i2):
        pltpu.make_async_copy(xf_hbm.at[idxj_v.at[i3]], rows_v.at[i2],
                              semg[i2]).wait()

    def _mul_scatter(i3, i2):
        def _mrow(i, cc):
            for j in range(NF // 16):
                s = pl.ds(j * 16, 16)
                rows_v[i2, i, s] = rows_v[i2, i, s] * g_v[i2, i, s]
            return cc
        lax.fori_loop(0, CH, _mrow, 0)
        pltpu.sync_copy(rows_v.at[i2], acc.at[idxi_v.at[i3]], add=True)

    _issue_idx(ebase, 0)
    _issue_idx(ebase + CH, 1)
    _wait_idx(0)
    _issue_g(0, 0)
    _issue_gt(0, 0)

    def _group(g, carry):
        gbase = ebase + g * (NUNROLL * CH)
        for u in range(NUNROLL):
            i3, i3n = u % 3, (u + 1) % 3
            i2, i2n = u % 2, (u + 1) % 2
            kbase = gbase + u * CH
            if u < NUNROLL - 2:
                _issue_idx(kbase + 2 * CH, (u + 2) % 3)
            else:
                @pl.when(g < NGROUPS - 1)
                def _a():
                    _issue_idx(kbase + 2 * CH, (u + 2) % 3)
            if u < NUNROLL - 1:
                _wait_idx(i3n)
                _issue_g(i3n, i2n)
                _issue_gt(i3n, i2n)
            else:
                @pl.when(g < NGROUPS - 1)
                def _b():
                    _wait_idx(i3n)
                    _issue_g(i3n, i2n)
                    _issue_gt(i3n, i2n)
            _wait_g(i3, i2)
            _wait_gt(i3, i2)
            _mul_scatter(i3, i2)
        return carry
    lax.fori_loop(0, NGROUPS, _group, 0)

    @pl.when(sid < CH_REM)
    def _extra():
        base = cid * EPC + (SC_SUBCORES * CH_BASE + sid) * CH
        pltpu.sync_copy(idxj_hbm.at[pl.ds(base, CH)], idxj_v.at[0])
        pltpu.sync_copy(idxi_hbm.at[pl.ds(base, CH)], idxi_v.at[0])
        pltpu.sync_copy(idxk_hbm.at[pl.ds(base, CH)], idxk_v.at[0])
        pltpu.async_copy(xf_hbm.at[idxj_v.at[0]], rows_v.at[0],
                         semg[0]).wait()
        pltpu.async_copy(gt_hbm.at[idxk_v.at[0]], g_v.at[0],
                         semw[0]).wait()
        _mul_scatter(0, 0)

    plsc.subcore_barrier()

    @pl.when(sid == SC_SUBCORES - 1)
    def _flast():
        pltpu.sync_copy(acc.at[pl.ds(r0, RPT_LAST)],
                        out_hbm.at[pl.ds(cid * N + r0, RPT_LAST)])

    @pl.when(sid < SC_SUBCORES - 1)
    def _frest():
        pltpu.sync_copy(acc.at[pl.ds(r0, RPT)],
                        out_hbm.at[pl.ds(cid * N + r0, RPT)])


def _edge_pass(xf, g_table, idx_i, idx_j, idx_k):
    mesh = plsc.VectorSubcoreMesh(core_axis_name="c", subcore_axis_name="s")
    run = pl.kernel(
        _edge_body,
        out_type=jax.ShapeDtypeStruct((SC_CORES * N, NF), jnp.float32),
        mesh=mesh,
        scratch_types=[
            pltpu.VMEM_SHARED((N, NF), jnp.float32),
            pltpu.VMEM((3, CH), jnp.int32),
            pltpu.VMEM((3, CH), jnp.int32),
            pltpu.VMEM((3, CH), jnp.int32),
            pltpu.VMEM((2, CH, NF), jnp.float32),
            pltpu.VMEM((2, CH, NF), jnp.float32),
        ] + [pltpu.SemaphoreType.DMA] * 7,
    )
    out = run(xf, g_table, idx_i, idx_j, idx_k)
    return out.reshape(SC_CORES, N, NF)


def kernel(atomic_numbers, r_ij, idx_i, idx_j, emb, in2f_W, fn_W1, fn_b1,
           fn_W2, fn_b2, f2out_W1, f2out_b1, f2out_W2, f2out_b2):
    emb_pad = jnp.pad(emb, ((0, NAB - MAXZ), (0, 0)))
    idx_i = idx_i.astype(jnp.int32)
    idx_j = idx_j.astype(jnp.int32)

    idx_k = _prep(r_ij)
    x, xf = _init(atomic_numbers, emb_pad, in2f_W[0])
    for t in range(NI):
        g_t = _filter_table(fn_W1[t], fn_b1[t], fn_W2[t], fn_b2[t])
        partials = _edge_pass(xf, g_t, idx_i, idx_j, idx_k)
        nxt = in2f_W[t + 1] if t + 1 < NI else None
        x, xf = _update(partials, x, f2out_W1[t], f2out_b1[t],
                        f2out_W2[t], f2out_b2[t], nxt)
    return x


# speedup vs baseline: 5.3052x; 1.0449x over previous
import functools
import math

import jax
import jax.numpy as jnp
from jax import lax
from jax.experimental import pallas as pl
from jax.experimental.pallas import tpu as pltpu
from jax.experimental.pallas import tpu_sc as plsc

N = 10000
E = 320000
NAB = 128
NF = 128
NI = 3
NRBF = 20
CUTOFF = 5.0
MAXZ = 100

_LN2 = math.log(2.0)


def _ssp(x):
    return jnp.maximum(x, 0.0) + jnp.log(1.0 + jnp.exp(-jnp.abs(x))) - _LN2


TKNOT = 16384
TPAD = 2048
TROWS = TKNOT + TPAD
TB = 2048
DH = CUTOFF / TKNOT


def _table_body(w1_ref, b1_ref, w2_ref, b2_ref, out_ref):
    i = pl.program_id(0)
    row = lax.broadcasted_iota(jnp.int32, (TB, 1), 0) + i * TB
    d = row.astype(jnp.float32) * DH
    width = CUTOFF / (NRBF - 1)
    offs = lax.broadcasted_iota(jnp.int32, (1, NRBF), 1).astype(jnp.float32)
    offs = offs * width
    coeff = -0.5 / (width * width)
    f = jnp.exp(coeff * (d - offs) ** 2)
    rcut = 0.5 * (jnp.cos(d * (math.pi / CUTOFF)) + 1.0)
    rcut = rcut * (d < CUTOFF).astype(jnp.float32)
    hid = _ssp(jnp.dot(f, w1_ref[...], preferred_element_type=jnp.float32)
               + b1_ref[...])
    w = (jnp.dot(hid, w2_ref[...], preferred_element_type=jnp.float32)
         + b2_ref[...])
    out_ref[...] = w * rcut


def _filter_table(fn_w1, fn_b1, fn_w2, fn_b2):
    return pl.pallas_call(
        _table_body,
        grid=(TROWS // TB,),
        in_specs=[
            pl.BlockSpec((NRBF, NF), lambda i: (0, 0)),
            pl.BlockSpec((1, NF), lambda i: (0, 0)),
            pl.BlockSpec((NF, NF), lambda i: (0, 0)),
            pl.BlockSpec((1, NF), lambda i: (0, 0)),
        ],
        out_specs=pl.BlockSpec((TB, NF), lambda i: (i, 0)),
        out_shape=jax.ShapeDtypeStruct((TROWS, NF), jnp.float32),
    )(fn_w1, fn_b1.reshape(1, NF), fn_w2, fn_b2.reshape(1, NF))


BE = 2000


def _prep_body(r_ref, out_ref):
    r = r_ref[...]
    d = jnp.sqrt(r[:, 0:1] ** 2 + r[:, 1:2] ** 2 + r[:, 2:3] ** 2)
    k = (d * (1.0 / DH) + 0.5).astype(jnp.int32)
    out_ref[...] = jnp.minimum(k, TROWS - 1)


def _prep(r_ij):
    out = pl.pallas_call(
        _prep_body,
        grid=(E // BE,),
        in_specs=[pl.BlockSpec((BE, 3), lambda i: (i, 0))],
        out_specs=pl.BlockSpec((BE, 1), lambda i: (i, 0)),
        out_shape=jax.ShapeDtypeStruct((E, 1), jnp.int32),
    )(r_ij)
    return out.reshape(E)


BN = 2000


def _init_body(an_ref, emb_ref, w_ref, x_ref, xf_ref):
    ids = an_ref[0, 0, :].reshape(BN, 1)
    cols = lax.broadcasted_iota(jnp.int32, (BN, NAB), 1)
    onehot = (ids == cols).astype(jnp.float32)
    x = jnp.dot(onehot, emb_ref[...], preferred_element_type=jnp.float32)
    x_ref[...] = x
    xf_ref[...] = jnp.dot(x, w_ref[...], preferred_element_type=jnp.float32)


def _init(atomic_numbers, emb_pad, in2f_w0):
    an3 = atomic_numbers.astype(jnp.int32).reshape(N // BN, 1, BN)
    return pl.pallas_call(
        _init_body,
        grid=(N // BN,),
        in_specs=[
            pl.BlockSpec((1, 1, BN), lambda i: (i, 0, 0)),
            pl.BlockSpec((NAB, NAB), lambda i: (0, 0)),
            pl.BlockSpec((NAB, NF), lambda i: (0, 0)),
        ],
        out_specs=[
            pl.BlockSpec((BN, NAB), lambda i: (i, 0)),
            pl.BlockSpec((BN, NF), lambda i: (i, 0)),
        ],
        out_shape=[
            jax.ShapeDtypeStruct((N, NAB), jnp.float32),
            jax.ShapeDtypeStruct((N, NF), jnp.float32),
        ],
    )(an3, emb_pad, in2f_w0)


def _update_body(has_next, p_ref, x_ref, w1_ref, b1_ref, w2_ref, b2_ref,
                 wn_ref, x_out_ref, xf_out_ref=None):
    agg = p_ref[0] + p_ref[1]
    h = _ssp(jnp.dot(agg, w1_ref[...], preferred_element_type=jnp.float32)
             + b1_ref[...])
    v = jnp.dot(h, w2_ref[...], preferred_element_type=jnp.float32) + b2_ref[...]
    xn = x_ref[...] + v
    x_out_ref[...] = xn
    if has_next:
        xf_out_ref[...] = jnp.dot(xn, wn_ref[...],
                                  preferred_element_type=jnp.float32)


def _update(partials, x, w1, b1, w2, b2, in2f_next):
    has_next = in2f_next is not None
    wn = in2f_next if has_next else jnp.zeros((NAB, NF), jnp.float32)
    out_specs = [pl.BlockSpec((BN, NAB), lambda i: (i, 0))]
    out_shape = [jax.ShapeDtypeStruct((N, NAB), jnp.float32)]
    if has_next:
        out_specs.append(pl.BlockSpec((BN, NF), lambda i: (i, 0)))
        out_shape.append(jax.ShapeDtypeStruct((N, NF), jnp.float32))
    res = pl.pallas_call(
        functools.partial(_update_body, has_next),
        grid=(N // BN,),
        in_specs=[
            pl.BlockSpec((2, BN, NF), lambda i: (0, i, 0)),
            pl.BlockSpec((BN, NAB), lambda i: (i, 0)),
            pl.BlockSpec((NF, NAB), lambda i: (0, 0)),
            pl.BlockSpec((1, NAB), lambda i: (0, 0)),
            pl.BlockSpec((NAB, NAB), lambda i: (0, 0)),
            pl.BlockSpec((1, NAB), lambda i: (0, 0)),
            pl.BlockSpec((NAB, NF), lambda i: (0, 0)),
        ],
        out_specs=out_specs,
        out_shape=out_shape,
    )(partials, x, w1, b1.reshape(1, NAB), w2, b2.reshape(1, NAB), wn)
    if has_next:
        return res[0], res[1]
    return res[0], None


SC_CORES = 2
SC_SUBCORES = 16
CH = 64
EPC = E // SC_CORES
NCHC = EPC // CH
CH_BASE = NCHC // SC_SUBCORES
CH_REM = NCHC % SC_SUBCORES
RPT = 624
RPT_LAST = N - 15 * RPT
NUNROLL = 12
NGROUPS = CH_BASE // NUNROLL


def _edge_body(xf_hbm, gt_hbm, idxi_hbm, idxj_hbm, idxk_hbm, out_hbm,
               acc, idxi_v, idxj_v, idxk_v, g_v, rows_v, *sems):
    semi = sems[0:4]
    semw = sems[4:6]
    semg = sems[6:9]
    semsc = sems[9:12]
    cid = lax.axis_index("c")
    sid = lax.axis_index("s")

    zbuf = g_v.at[0]

    def _zrow(i, carry):
        for j in range(NF // 16):
            zbuf[i, pl.ds(j * 16, 16)] = jnp.zeros((16,), jnp.float32)
        return carry
    lax.fori_loop(0, CH, _zrow, 0)
    r0 = sid * RPT
    for k in range(RPT // CH):
        pltpu.sync_copy(zbuf, acc.at[pl.ds(r0 + k * CH, CH)])

    @pl.when(sid == SC_SUBCORES - 1)
    def _zlast():
        pltpu.sync_copy(zbuf, acc.at[pl.ds(r0 + (RPT // CH) * CH, CH)])

    @pl.when(sid < SC_SUBCORES - 1)
    def _zrest():
        zr = RPT - (RPT // CH) * CH
        pltpu.sync_copy(zbuf.at[pl.ds(0, zr)],
                        acc.at[pl.ds(r0 + (RPT // CH) * CH, zr)])
    plsc.subcore_barrier()

    ebase = cid * EPC + sid * (CH_BASE * CH)

    def _issue_idx(base, i4):
        pltpu.async_copy(idxj_hbm.at[pl.ds(base, CH)], idxj_v.at[i4],
                         semi[i4])
        pltpu.async_copy(idxi_hbm.at[pl.ds(base, CH)], idxi_v.at[i4],
                         semi[i4])
        pltpu.async_copy(idxk_hbm.at[pl.ds(base, CH)], idxk_v.at[i4],
                         semi[i4])

    def _wait_idx(i4):
        pltpu.make_async_copy(idxj_hbm.at[pl.ds(0, CH)], idxj_v.at[i4],
                              semi[i4]).wait()
        pltpu.make_async_copy(idxi_hbm.at[pl.ds(0, CH)], idxi_v.at[i4],
                              semi[i4]).wait()
        pltpu.make_async_copy(idxk_hbm.at[pl.ds(0, CH)], idxk_v.at[i4],
                              semi[i4]).wait()

    def _issue_gt(i4, i2):
        pltpu.async_copy(gt_hbm.at[idxk_v.at[i4]], g_v.at[i2], semw[i2])

    def _wait_gt(i4, i2):
        pltpu.make_async_copy(gt_hbm.at[idxk_v.at[i4]], g_v.at[i2],
                              semw[i2]).wait()

    def _issue_g(i4, i3):
        pltpu.async_copy(xf_hbm.at[idxj_v.at[i4]], rows_v.at[i3], semg[i3])

    def _wait_g(i4, i3):
        pltpu.make_async_copy(xf_hbm.at[idxj_v.at[i4]], rows_v.at[i3],
                              semg[i3]).wait()

    def _issue_sc(i4, i3):
        pltpu.async_copy(rows_v.at[i3], acc.at[idxi_v.at[i4]], semsc[i3],
                         add=True)

    def _wait_sc(i4, i3):
        pltpu.make_async_copy(rows_v.at[i3], acc.at[idxi_v.at[i4]],
                              semsc[i3]).wait()

    def _mul(i3, i2):
        def _mrow(q, cc):
            for di in range(4):
                i = q * 4 + di
                for j in range(NF // 16):
                    s = pl.ds(j * 16, 16)
                    rows_v[i3, i, s] = rows_v[i3, i, s] * g_v[i2, i, s]
            return cc
        lax.fori_loop(0, CH // 4, _mrow, 0)

    _issue_idx(ebase, 0)
    _issue_idx(ebase + CH, 1)
    _wait_idx(0)
    _issue_g(0, 0)
    _issue_gt(0, 0)

    def _group(g, carry):
        gbase = ebase + g * (NUNROLL * CH)
        for u in range(NUNROLL):
            i4, i4n, i4nn = u % 4, (u + 1) % 4, (u + 2) % 4
            i3, i3n = u % 3, (u + 1) % 3
            i2, i2n = u % 2, (u + 1) % 2
            kbase = gbase + u * CH
            if u >= 2:
                _wait_sc((u - 2) % 4, i3n)
            else:
                @pl.when(g > 0)
                def _w0():
                    _wait_sc((u + 2) % 4, i3n)
            if u < NUNROLL - 1:
                _wait_idx(i4n)
                _issue_g(i4n, i3n)
                _issue_gt(i4n, i2n)
            else:
                @pl.when(g < NGROUPS - 1)
                def _b():
                    _wait_idx(i4n)
                    _issue_g(i4n, i3n)
                    _issue_gt(i4n, i2n)
            if u < NUNROLL - 2:
                _issue_idx(kbase + 2 * CH, i4nn)
            else:
                @pl.when(g < NGROUPS - 1)
                def _a():
                    _issue_idx(kbase + 2 * CH, i4nn)
            _wait_g(i4, i3)
            _wait_gt(i4, i2)
            _mul(i3, i2)
            _issue_sc(i4, i3)
        return carry
    lax.fori_loop(0, NGROUPS, _group, 0)

    _wait_sc((CH_BASE - 2) % 4, (CH_BASE - 2) % 3)
    _wait_sc((CH_BASE - 1) % 4, (CH_BASE - 1) % 3)

    @pl.when(sid < CH_REM)
    def _extra():
        base = cid * EPC + (SC_SUBCORES * CH_BASE + sid) * CH
        pltpu.sync_copy(idxj_hbm.at[pl.ds(base, CH)], idxj_v.at[0])
        pltpu.sync_copy(idxi_hbm.at[pl.ds(base, CH)], idxi_v.at[0])
        pltpu.sync_copy(idxk_hbm.at[pl.ds(base, CH)], idxk_v.at[0])
        pltpu.async_copy(xf_hbm.at[idxj_v.at[0]], rows_v.at[0],
                         semg[0]).wait()
        pltpu.async_copy(gt_hbm.at[idxk_v.at[0]], g_v.at[0],
                         semw[0]).wait()
        _mul(0, 0)
        pltpu.sync_copy(rows_v.at[0], acc.at[idxi_v.at[0]], add=True)

    plsc.subcore_barrier()

    @pl.when(sid == SC_SUBCORES - 1)
    def _flast():
        pltpu.sync_copy(acc.at[pl.ds(r0, RPT_LAST)],
                        out_hbm.at[pl.ds(cid * N + r0, RPT_LAST)])

    @pl.when(sid < SC_SUBCORES - 1)
    def _frest():
        pltpu.sync_copy(acc.at[pl.ds(r0, RPT)],
                        out_hbm.at[pl.ds(cid * N + r0, RPT)])


def _edge_pass(xf, g_table, idx_i, idx_j, idx_k):
    mesh = plsc.VectorSubcoreMesh(core_axis_name="c", subcore_axis_name="s")
    run = pl.kernel(
        _edge_body,
        out_type=jax.ShapeDtypeStruct((SC_CORES * N, NF), jnp.float32),
        mesh=mesh,
        scratch_types=[
            pltpu.VMEM_SHARED((N, NF), jnp.float32),
            pltpu.VMEM((4, CH), jnp.int32),
            pltpu.VMEM((4, CH), jnp.int32),
            pltpu.VMEM((4, CH), jnp.int32),
            pltpu.VMEM((2, CH, NF), jnp.float32),
            pltpu.VMEM((3, CH, NF), jnp.float32),
        ] + [pltpu.SemaphoreType.DMA] * 12,
    )
    out = run(xf, g_table, idx_i, idx_j, idx_k)
    return out.reshape(SC_CORES, N, NF)


def kernel(atomic_numbers, r_ij, idx_i, idx_j, emb, in2f_W, fn_W1, fn_b1,
           fn_W2, fn_b2, f2out_W1, f2out_b1, f2out_W2, f2out_b2):
    emb_pad = jnp.pad(emb, ((0, NAB - MAXZ), (0, 0)))
    idx_i = idx_i.astype(jnp.int32)
    idx_j = idx_j.astype(jnp.int32)

    idx_k = _prep(r_ij)
    x, xf = _init(atomic_numbers, emb_pad, in2f_W[0])
    for t in range(NI):
        g_t = _filter_table(fn_W1[t], fn_b1[t], fn_W2[t], fn_b2[t])
        partials = _edge_pass(xf, g_t, idx_i, idx_j, idx_k)
        nxt = in2f_W[t + 1] if t + 1 < NI else None
        x, xf = _update(partials, x, f2out_W1[t], f2out_b1[t],
                        f2out_W2[t], f2out_b2[t], nxt)
    return x


# speedup vs baseline: 7.5871x; 1.4301x over previous
import functools
import math

import jax
import jax.numpy as jnp
from jax import lax
from jax.experimental import pallas as pl
from jax.experimental.pallas import tpu as pltpu
from jax.experimental.pallas import tpu_sc as plsc

N = 10000
E = 320000
NAB = 128
NF = 128
NI = 3
NRBF = 20
CUTOFF = 5.0
MAXZ = 100

_LN2 = math.log(2.0)


def _ssp(x):
    return jnp.maximum(x, 0.0) + jnp.log(1.0 + jnp.exp(-jnp.abs(x))) - _LN2


TKNOT = 16384
TPAD = 2048
TROWS = TKNOT + TPAD
TB = 2048
DH = CUTOFF / TKNOT


def _table_body(w1_ref, b1_ref, w2_ref, b2_ref, out_ref):
    i = pl.program_id(0)
    row = lax.broadcasted_iota(jnp.int32, (TB, 1), 0) + i * TB
    d = row.astype(jnp.float32) * DH
    width = CUTOFF / (NRBF - 1)
    offs = lax.broadcasted_iota(jnp.int32, (1, NRBF), 1).astype(jnp.float32)
    offs = offs * width
    coeff = -0.5 / (width * width)
    f = jnp.exp(coeff * (d - offs) ** 2)
    rcut = 0.5 * (jnp.cos(d * (math.pi / CUTOFF)) + 1.0)
    rcut = rcut * (d < CUTOFF).astype(jnp.float32)
    hid = _ssp(jnp.dot(f, w1_ref[...], preferred_element_type=jnp.float32)
               + b1_ref[...])
    w = (jnp.dot(hid, w2_ref[...], preferred_element_type=jnp.float32)
         + b2_ref[...])
    out_ref[...] = w * rcut


def _filter_table(fn_w1, fn_b1, fn_w2, fn_b2):
    return pl.pallas_call(
        _table_body,
        grid=(TROWS // TB,),
        in_specs=[
            pl.BlockSpec((NRBF, NF), lambda i: (0, 0)),
            pl.BlockSpec((1, NF), lambda i: (0, 0)),
            pl.BlockSpec((NF, NF), lambda i: (0, 0)),
            pl.BlockSpec((1, NF), lambda i: (0, 0)),
        ],
        out_specs=pl.BlockSpec((TB, NF), lambda i: (i, 0)),
        out_shape=jax.ShapeDtypeStruct((TROWS, NF), jnp.float32),
    )(fn_w1, fn_b1.reshape(1, NF), fn_w2, fn_b2.reshape(1, NF))


CHP = 512
NCHP = E // CHP


def _prep_body(xs_hbm, ys_hbm, zs_hbm, out_hbm, xv, yv, zv, kv, sem):
    cid = lax.axis_index("c")
    sid = lax.axis_index("s")
    wid = sid * SC_CORES + cid

    def _chunk(i, carry):
        ch = wid + 32 * i

        @pl.when(ch < NCHP)
        def _do():
            base = ch * CHP
            pltpu.sync_copy(xs_hbm.at[pl.ds(base, CHP)], xv)
            pltpu.sync_copy(ys_hbm.at[pl.ds(base, CHP)], yv)
            pltpu.sync_copy(zs_hbm.at[pl.ds(base, CHP)], zv)

            def _grp(g, c2):
                s = pl.ds(g * 16, 16)
                x = xv[s]
                y = yv[s]
                z = zv[s]
                q = x * x + y * y + z * z
                qi = lax.bitcast_convert_type(q, jnp.int32)
                w = lax.bitcast_convert_type(
                    jnp.int32(0x5F3759DF) - lax.div(qi, jnp.int32(2)),
                    jnp.float32)
                w = w * (1.5 - 0.5 * q * w * w)
                w = w * (1.5 - 0.5 * q * w * w)
                w = w * (1.5 - 0.5 * q * w * w)
                d = q * w
                k = (d * (1.0 / DH) + 0.5).astype(jnp.int32)
                kv[pl.ds(g * 16, 16)] = jnp.minimum(k, TROWS - 1)
                return c2
            lax.fori_loop(0, CHP // 16, _grp, 0)
            pltpu.sync_copy(kv, out_hbm.at[pl.ds(base, CHP)])
        return carry
    lax.fori_loop(0, (NCHP + 31) // 32, _chunk, 0)


def _prep(r_ij):
    mesh = plsc.VectorSubcoreMesh(core_axis_name="c", subcore_axis_name="s")
    run = pl.kernel(
        _prep_body,
        out_type=jax.ShapeDtypeStruct((E,), jnp.int32),
        mesh=mesh,
        scratch_types=[
            pltpu.VMEM((CHP,), jnp.float32),
            pltpu.VMEM((CHP,), jnp.float32),
            pltpu.VMEM((CHP,), jnp.float32),
            pltpu.VMEM((CHP,), jnp.int32),
            pltpu.SemaphoreType.DMA,
        ],
    )
    return run(r_ij[:, 0], r_ij[:, 1], r_ij[:, 2])


BN = 2000


def _init_body(an_ref, emb_ref, w_ref, x_ref, xf_ref):
    ids = an_ref[0, 0, :].reshape(BN, 1)
    cols = lax.broadcasted_iota(jnp.int32, (BN, NAB), 1)
    onehot = (ids == cols).astype(jnp.float32)
    x = jnp.dot(onehot, emb_ref[...], preferred_element_type=jnp.float32)
    x_ref[...] = x
    xf_ref[...] = jnp.dot(x, w_ref[...], preferred_element_type=jnp.float32)


def _init(atomic_numbers, emb_pad, in2f_w0):
    an3 = atomic_numbers.astype(jnp.int32).reshape(N // BN, 1, BN)
    return pl.pallas_call(
        _init_body,
        grid=(N // BN,),
        in_specs=[
            pl.BlockSpec((1, 1, BN), lambda i: (i, 0, 0)),
            pl.BlockSpec((NAB, NAB), lambda i: (0, 0)),
            pl.BlockSpec((NAB, NF), lambda i: (0, 0)),
        ],
        out_specs=[
            pl.BlockSpec((BN, NAB), lambda i: (i, 0)),
            pl.BlockSpec((BN, NF), lambda i: (i, 0)),
        ],
        out_shape=[
            jax.ShapeDtypeStruct((N, NAB), jnp.float32),
            jax.ShapeDtypeStruct((N, NF), jnp.float32),
        ],
    )(an3, emb_pad, in2f_w0)


def _update_body(has_next, p_ref, x_ref, w1_ref, b1_ref, w2_ref, b2_ref,
                 wn_ref, x_out_ref, xf_out_ref=None):
    agg = p_ref[0] + p_ref[1]
    h = _ssp(jnp.dot(agg, w1_ref[...], preferred_element_type=jnp.float32)
             + b1_ref[...])
    v = jnp.dot(h, w2_ref[...], preferred_element_type=jnp.float32) + b2_ref[...]
    xn = x_ref[...] + v
    x_out_ref[...] = xn
    if has_next:
        xf_out_ref[...] = jnp.dot(xn, wn_ref[...],
                                  preferred_element_type=jnp.float32)


def _update(partials, x, w1, b1, w2, b2, in2f_next):
    has_next = in2f_next is not None
    wn = in2f_next if has_next else jnp.zeros((NAB, NF), jnp.float32)
    out_specs = [pl.BlockSpec((BN, NAB), lambda i: (i, 0))]
    out_shape = [jax.ShapeDtypeStruct((N, NAB), jnp.float32)]
    if has_next:
        out_specs.append(pl.BlockSpec((BN, NF), lambda i: (i, 0)))
        out_shape.append(jax.ShapeDtypeStruct((N, NF), jnp.float32))
    res = pl.pallas_call(
        functools.partial(_update_body, has_next),
        grid=(N // BN,),
        in_specs=[
            pl.BlockSpec((2, BN, NF), lambda i: (0, i, 0)),
            pl.BlockSpec((BN, NAB), lambda i: (i, 0)),
            pl.BlockSpec((NF, NAB), lambda i: (0, 0)),
            pl.BlockSpec((1, NAB), lambda i: (0, 0)),
            pl.BlockSpec((NAB, NAB), lambda i: (0, 0)),
            pl.BlockSpec((1, NAB), lambda i: (0, 0)),
            pl.BlockSpec((NAB, NF), lambda i: (0, 0)),
        ],
        out_specs=out_specs,
        out_shape=out_shape,
    )(partials, x, w1, b1.reshape(1, NAB), w2, b2.reshape(1, NAB), wn)
    if has_next:
        return res[0], res[1]
    return res[0], None


SC_CORES = 2
SC_SUBCORES = 16
CH = 64
EPC = E // SC_CORES
NCHC = EPC // CH
CH_BASE = NCHC // SC_SUBCORES
CH_REM = NCHC % SC_SUBCORES
RPT = 624
RPT_LAST = N - 15 * RPT
NUNROLL = 12
NGROUPS = CH_BASE // NUNROLL


def _edge_body(xf_hbm, gt_hbm, idxi_hbm, idxj_hbm, idxk_hbm, out_hbm,
               acc, idxi_v, idxj_v, idxk_v, g_v, rows_v, *sems):
    semi = sems[0:4]
    semw = sems[4:6]
    semg = sems[6:9]
    semsc = sems[9:12]
    cid = lax.axis_index("c")
    sid = lax.axis_index("s")

    zbuf = g_v.at[0]

    def _zrow(i, carry):
        for j in range(NF // 16):
            zbuf[i, pl.ds(j * 16, 16)] = jnp.zeros((16,), jnp.float32)
        return carry
    lax.fori_loop(0, CH, _zrow, 0)
    r0 = sid * RPT
    for k in range(RPT // CH):
        pltpu.sync_copy(zbuf, acc.at[pl.ds(r0 + k * CH, CH)])

    @pl.when(sid == SC_SUBCORES - 1)
    def _zlast():
        pltpu.sync_copy(zbuf, acc.at[pl.ds(r0 + (RPT // CH) * CH, CH)])

    @pl.when(sid < SC_SUBCORES - 1)
    def _zrest():
        zr = RPT - (RPT // CH) * CH
        pltpu.sync_copy(zbuf.at[pl.ds(0, zr)],
                        acc.at[pl.ds(r0 + (RPT // CH) * CH, zr)])
    plsc.subcore_barrier()

    ebase = cid * EPC + sid * (CH_BASE * CH)

    def _issue_idx(base, i4):
        pltpu.async_copy(idxj_hbm.at[pl.ds(base, CH)], idxj_v.at[i4],
                         semi[i4])
        pltpu.async_copy(idxi_hbm.at[pl.ds(base, CH)], idxi_v.at[i4],
                         semi[i4])
        pltpu.async_copy(idxk_hbm.at[pl.ds(base, CH)], idxk_v.at[i4],
                         semi[i4])

    def _wait_idx(i4):
        pltpu.make_async_copy(idxj_hbm.at[pl.ds(0, CH)], idxj_v.at[i4],
                              semi[i4]).wait()
        pltpu.make_async_copy(idxi_hbm.at[pl.ds(0, CH)], idxi_v.at[i4],
                              semi[i4]).wait()
        pltpu.make_async_copy(idxk_hbm.at[pl.ds(0, CH)], idxk_v.at[i4],
                              semi[i4]).wait()

    def _issue_gt(i4, i2):
        pltpu.async_copy(gt_hbm.at[idxk_v.at[i4]], g_v.at[i2], semw[i2])

    def _wait_gt(i4, i2):
        pltpu.make_async_copy(gt_hbm.at[idxk_v.at[i4]], g_v.at[i2],
                              semw[i2]).wait()

    def _issue_g(i4, i3):
        pltpu.async_copy(xf_hbm.at[idxj_v.at[i4]], rows_v.at[i3], semg[i3])

    def _wait_g(i4, i3):
        pltpu.make_async_copy(xf_hbm.at[idxj_v.at[i4]], rows_v.at[i3],
                              semg[i3]).wait()

    def _issue_sc(i4, i3):
        pltpu.async_copy(rows_v.at[i3], acc.at[idxi_v.at[i4]], semsc[i3],
                         add=True)

    def _wait_sc(i4, i3):
        pltpu.make_async_copy(rows_v.at[i3], acc.at[idxi_v.at[i4]],
                              semsc[i3]).wait()

    def _mul(i3, i2):
        def _mrow(q, cc):
            for di in range(4):
                i = q * 4 + di
                for j in range(NF // 16):
                    s = pl.ds(j * 16, 16)
                    rows_v[i3, i, s] = rows_v[i3, i, s] * g_v[i2, i, s]
            return cc
        lax.fori_loop(0, CH // 4, _mrow, 0)

    _issue_idx(ebase, 0)
    _issue_idx(ebase + CH, 1)
    _wait_idx(0)
    _issue_g(0, 0)
    _issue_gt(0, 0)

    def _group(g, carry):
        gbase = ebase + g * (NUNROLL * CH)
        for u in range(NUNROLL):
            i4, i4n, i4nn = u % 4, (u + 1) % 4, (u + 2) % 4
            i3, i3n = u % 3, (u + 1) % 3
            i2, i2n = u % 2, (u + 1) % 2
            kbase = gbase + u * CH
            if u >= 2:
                _wait_sc((u - 2) % 4, i3n)
            else:
                @pl.when(g > 0)
                def _w0():
                    _wait_sc((u + 2) % 4, i3n)
            if u < NUNROLL - 1:
                _wait_idx(i4n)
                _issue_g(i4n, i3n)
                _issue_gt(i4n, i2n)
            else:
                @pl.when(g < NGROUPS - 1)
                def _b():
                    _wait_idx(i4n)
                    _issue_g(i4n, i3n)
                    _issue_gt(i4n, i2n)
            if u < NUNROLL - 2:
                _issue_idx(kbase + 2 * CH, i4nn)
            else:
                @pl.when(g < NGROUPS - 1)
                def _a():
                    _issue_idx(kbase + 2 * CH, i4nn)
            _wait_g(i4, i3)
            _wait_gt(i4, i2)
            _mul(i3, i2)
            _issue_sc(i4, i3)
        return carry
    lax.fori_loop(0, NGROUPS, _group, 0)

    _wait_sc((CH_BASE - 2) % 4, (CH_BASE - 2) % 3)
    _wait_sc((CH_BASE - 1) % 4, (CH_BASE - 1) % 3)

    @pl.when(sid < CH_REM)
    def _extra():
        base = cid * EPC + (SC_SUBCORES * CH_BASE + sid) * CH
        pltpu.sync_copy(idxj_hbm.at[pl.ds(base, CH)], idxj_v.at[0])
        pltpu.sync_copy(idxi_hbm.at[pl.ds(base, CH)], idxi_v.at[0])
        pltpu.sync_copy(idxk_hbm.at[pl.ds(base, CH)], idxk_v.at[0])
        pltpu.async_copy(xf_hbm.at[idxj_v.at[0]], rows_v.at[0],
                         semg[0]).wait()
        pltpu.async_copy(gt_hbm.at[idxk_v.at[0]], g_v.at[0],
                         semw[0]).wait()
        _mul(0, 0)
        pltpu.sync_copy(rows_v.at[0], acc.at[idxi_v.at[0]], add=True)

    plsc.subcore_barrier()

    @pl.when(sid == SC_SUBCORES - 1)
    def _flast():
        pltpu.sync_copy(acc.at[pl.ds(r0, RPT_LAST)],
                        out_hbm.at[pl.ds(cid * N + r0, RPT_LAST)])

    @pl.when(sid < SC_SUBCORES - 1)
    def _frest():
        pltpu.sync_copy(acc.at[pl.ds(r0, RPT)],
                        out_hbm.at[pl.ds(cid * N + r0, RPT)])


def _edge_pass(xf, g_table, idx_i, idx_j, idx_k):
    mesh = plsc.VectorSubcoreMesh(core_axis_name="c", subcore_axis_name="s")
    run = pl.kernel(
        _edge_body,
        out_type=jax.ShapeDtypeStruct((SC_CORES * N, NF), jnp.float32),
        mesh=mesh,
        scratch_types=[
            pltpu.VMEM_SHARED((N, NF), jnp.float32),
            pltpu.VMEM((4, CH), jnp.int32),
            pltpu.VMEM((4, CH), jnp.int32),
            pltpu.VMEM((4, CH), jnp.int32),
            pltpu.VMEM((2, CH, NF), jnp.float32),
            pltpu.VMEM((3, CH, NF), jnp.float32),
        ] + [pltpu.SemaphoreType.DMA] * 12,
    )
    out = run(xf, g_table, idx_i, idx_j, idx_k)
    return out.reshape(SC_CORES, N, NF)


def kernel(atomic_numbers, r_ij, idx_i, idx_j, emb, in2f_W, fn_W1, fn_b1,
           fn_W2, fn_b2, f2out_W1, f2out_b1, f2out_W2, f2out_b2):
    emb_pad = jnp.pad(emb, ((0, NAB - MAXZ), (0, 0)))
    idx_i = idx_i.astype(jnp.int32)
    idx_j = idx_j.astype(jnp.int32)

    idx_k = _prep(r_ij)
    x, xf = _init(atomic_numbers, emb_pad, in2f_W[0])
    for t in range(NI):
        g_t = _filter_table(fn_W1[t], fn_b1[t], fn_W2[t], fn_b2[t])
        partials = _edge_pass(xf, g_t, idx_i, idx_j, idx_k)
        nxt = in2f_W[t + 1] if t + 1 < NI else None
        x, xf = _update(partials, x, f2out_W1[t], f2out_b1[t],
                        f2out_W2[t], f2out_b2[t], nxt)
    return x
